# baseline (speedup 1.0000x reference)
import jax
import jax.numpy as jnp
from jax.experimental import pallas as pl

B, T, S, C = 8, 8, 8, 64
S3 = S ** 3
NP = B // 2
NR = NP * S3
C2 = 2 * C
L = T - 1
BL = B * L
NH, DH = 4, 16
N_TF = 2


def _layernorm(x, g, b, eps=1e-5):
    J = jnp.full((C, C), 1.0 / C, jnp.float32)
    mu = jnp.dot(x, J, preferred_element_type=jnp.float32)
    s2 = jnp.dot(x * x, J, preferred_element_type=jnp.float32)
    var = s2 - mu * mu
    return (x - mu) * jax.lax.rsqrt(var + eps) * g + b


def _layernorm2(x, g2, b2, eps=1e-5):
    l1 = jax.lax.broadcasted_iota(jnp.int32, (C2, C2), 0)
    l2 = jax.lax.broadcasted_iota(jnp.int32, (C2, C2), 1)
    J = (l1 // C == l2 // C).astype(jnp.float32) * (1.0 / C)
    mu = jnp.dot(x, J, preferred_element_type=jnp.float32)
    s2 = jnp.dot(x * x, J, preferred_element_type=jnp.float32)
    var = s2 - mu * mu
    return (x - mu) * jax.lax.rsqrt(var + eps) * g2 + b2


def _body(t0_ref, acts_ref, ss_ref, ss4_ref, *refs):
    out_ref = refs[-1]
    vals = [r[...] for r in refs[:-1]]
    (lin2_W, lin2_b,
     conv0_Wl, conv0_bl, conv0_Wr,
     conv1_Wl, conv1_bl, conv1_Wr,
     norm_g, norm_b,
     act_lin_W, act_lin_b,
     scalar_W, scalar_b) = vals[:14]
    tf = vals[14:]

    t0_2 = t0_ref[...]
    acts = acts_ref[...]
    ss = ss_ref[...]
    ss4 = ss4_ref[...]

    lane = jax.lax.broadcasted_iota(jnp.int32, (NR, C2), 1)
    t0a = jnp.broadcast_to(t0_2[:, 0:1], (NR, C2))
    t0b = jnp.broadcast_to(t0_2[:, 1:2], (NR, C2))
    t0full = jnp.where(lane < C, t0a, t0b)

    rowidx = jax.lax.broadcasted_iota(jnp.int32, (NR, 1), 0)
    sid = rowidx % S3

    pr = jax.lax.broadcasted_iota(jnp.int32, (NP, NR), 0)
    pc = jax.lax.broadcasted_iota(jnp.int32, (NP, NR), 1)
    P = (pr == pc // S3).astype(jnp.float32)

    def bcast_rows(y):
        k = y.shape[1]
        return jnp.broadcast_to(y[:, None, :], (NP, S3, k)).reshape(NR, k)

    maskf_raw = (t0full != 0.0).astype(jnp.float32)
    cnt4 = jnp.dot(P, maskf_raw, preferred_element_type=jnp.float32)
    cnt_rows = bcast_rows(cnt4)
    maskf = jnp.where(jnp.logical_and(cnt_rows == 0.0, sid == 0),
                      1.0, maskf_raw)
    m_rows = jnp.maximum(cnt_rows, 1.0)
    deg = jnp.maximum(maskf * (m_rows - maskf), 1.0)
    md = maskf / deg

    inv_s1 = 1.0 / (S - 1)
    sid512 = jax.lax.broadcasted_iota(jnp.int32, (S3, 1), 0)
    ii = (sid512 // (S * S)).astype(jnp.float32) * inv_s1
    jj = ((sid512 // S) % S).astype(jnp.float32) * inv_s1
    kk = (sid512 % S).astype(jnp.float32) * inv_s1
    base = (ii * lin2_W[0:1, :] + jj * lin2_W[1:2, :] + kk * lin2_W[2:3, :]
            + lin2_b)
    baseN = jnp.broadcast_to(base[None], (NP, S3, C2)).reshape(NR, C2)
    ss_rows = bcast_rows(ss4)
    x = (baseN + (t0full * 0.5) * lin2_W[3:4, :]
         + (ss_rows * (1.0 / S)) * lin2_W[4:5, :])

    for Wl, bl, Wr in ((conv0_Wl, conv0_bl, conv0_Wr),
                       (conv1_Wl, conv1_bl, conv1_Wr)):
        Sx4 = jnp.dot(P, maskf * x, preferred_element_type=jnp.float32)
        SxRows = bcast_rows(Sx4)
        agg = md * (SxRows - x)
        h = (jnp.dot(agg, Wl, preferred_element_type=jnp.float32) + bl
             + jnp.dot(x, Wr, preferred_element_type=jnp.float32))
        x = _layernorm2(jnp.maximum(h, 0.0), norm_g, norm_b)

    full = x * maskf

    r2 = jax.lax.broadcasted_iota(jnp.int32, (24, S3), 0)
    c2 = jax.lax.broadcasted_iota(jnp.int32, (24, S3), 1)
    sel = jnp.where(r2 < 8, c2 // (S * S), jnp.where(r2 < 16, (c2 // S) % S, c2 % S))
    poolW = (sel == (r2 % 8)).astype(jnp.float32)

    a = jnp.dot(acts, act_lin_W, preferred_element_type=jnp.float32) + act_lin_b
    br = jax.lax.broadcasted_iota(jnp.int32, (BL, BL), 0)
    bc = jax.lax.broadcasted_iota(jnp.int32, (BL, BL), 1)
    blockmask = (br // L == bc // L).astype(jnp.float32)
    inv_sqrt_dh = 1.0 / 4.0
    for l in range(N_TF):
        (ln1_g, ln1_b, Wq, bq, Wk, bk, Wv, bv, Wo, bo,
         ln2_g, ln2_b, W1, b1, W2, b2) = tf[l * 16:(l + 1) * 16]
        xa = _layernorm(a, ln1_g, ln1_b)
        q = jnp.dot(xa, Wq, preferred_element_type=jnp.float32) + bq
        k = jnp.dot(xa, Wk, preferred_element_type=jnp.float32) + bk
        v = jnp.dot(xa, Wv, preferred_element_type=jnp.float32) + bv
        outs = []
        for hh in range(NH):
            qh = q[:, hh * DH:(hh + 1) * DH]
            kh = k[:, hh * DH:(hh + 1) * DH]
            vh = v[:, hh * DH:(hh + 1) * DH]
            sc = jax.lax.dot_general(
                qh, kh, (((1,), (1,)), ((), ())),
                preferred_element_type=jnp.float32) * inv_sqrt_dh
            sc = sc - jnp.max(sc, axis=-1, keepdims=True)
            e = jnp.exp(sc) * blockmask
            att = e / jnp.sum(e, axis=-1, keepdims=True)
            outs.append(jnp.dot(att, vh, preferred_element_type=jnp.float32))
        o = jnp.concatenate(outs, axis=1)
        a = a + jnp.dot(o, Wo, preferred_element_type=jnp.float32) + bo
        h2 = _layernorm(a, ln2_g, ln2_b)
        ff = jnp.maximum(
            jnp.dot(h2, W1, preferred_element_type=jnp.float32) + b1, 0.0)
        a = a + jnp.dot(ff, W2, preferred_element_type=jnp.float32) + b2

    ar = jax.lax.broadcasted_iota(jnp.int32, (B, BL), 0)
    ac = jax.lax.broadcasted_iota(jnp.int32, (B, BL), 1)
    Pact = (ar == ac // L).astype(jnp.float32) * (1.0 / L)
    act_emb = jnp.dot(Pact, a, preferred_element_type=jnp.float32)

    mv_emb = jnp.maximum(ss * scalar_W + scalar_b, 0.0)

    for k in range(NP):
        pooled = jnp.dot(poolW, full[k * S3:(k + 1) * S3, :],
                         preferred_element_type=jnp.float32) * (1.0 / 64.0)
        b0, b1 = 2 * k, 2 * k + 1
        out_ref[b0 * 26:b0 * 26 + 24, :] = pooled[:, :C]
        out_ref[b1 * 26:b1 * 26 + 24, :] = pooled[:, C:]
    for b in range(B):
        out_ref[b * 26 + 24:b * 26 + 25, :] = act_emb[b:b + 1, :]
        out_ref[b * 26 + 25:b * 26 + 26, :] = mv_emb[b:b + 1, :]


def kernel(xx, ss, lin_in_W, lin_in_b, conv0_Wl, conv0_bl, conv0_Wr,
           conv1_Wl, conv1_bl, conv1_Wr, norm_g, norm_b,
           act_lin_W, act_lin_b, scalar_W, scalar_b,
           tf0_ln1_g, tf0_ln1_b, tf0_Wq, tf0_bq, tf0_Wk, tf0_bk,
           tf0_Wv, tf0_bv, tf0_Wo, tf0_bo, tf0_ln2_g, tf0_ln2_b,
           tf0_W1, tf0_b1, tf0_W2, tf0_b2,
           tf1_ln1_g, tf1_ln1_b, tf1_Wq, tf1_bq, tf1_Wk, tf1_bk,
           tf1_Wv, tf1_bv, tf1_Wo, tf1_bo, tf1_ln2_g, tf1_ln2_b,
           tf1_W1, tf1_b1, tf1_W2, tf1_b2):
    t0 = xx[:, 0].reshape(B, S3).astype(jnp.float32)
    t0_2 = t0.reshape(NP, 2, S3).transpose(0, 2, 1).reshape(NR, 2)
    actsf = xx[:, 1:].reshape(BL, S3).astype(jnp.float32)
    ss4 = jnp.repeat(ss.reshape(NP, 2), C, axis=1)

    eye2 = jnp.eye(2, dtype=jnp.float32)

    def bd(W):
        return jnp.kron(eye2, W)

    def t2(v):
        return jnp.concatenate([v, v]).reshape(1, C2)

    def r2d(v):
        return v.reshape(1, -1)

    weights = [
        jnp.tile(lin_in_W, (1, 2)), t2(lin_in_b),
        bd(conv0_Wl), t2(conv0_bl), bd(conv0_Wr),
        bd(conv1_Wl), t2(conv1_bl), bd(conv1_Wr),
        t2(norm_g), t2(norm_b),
        act_lin_W, r2d(act_lin_b),
        scalar_W, r2d(scalar_b),
        r2d(tf0_ln1_g), r2d(tf0_ln1_b), tf0_Wq, r2d(tf0_bq), tf0_Wk, r2d(tf0_bk),
        tf0_Wv, r2d(tf0_bv), tf0_Wo, r2d(tf0_bo), r2d(tf0_ln2_g), r2d(tf0_ln2_b),
        tf0_W1, r2d(tf0_b1), tf0_W2, r2d(tf0_b2),
        r2d(tf1_ln1_g), r2d(tf1_ln1_b), tf1_Wq, r2d(tf1_bq), tf1_Wk, r2d(tf1_bk),
        tf1_Wv, r2d(tf1_bv), tf1_Wo, r2d(tf1_bo), r2d(tf1_ln2_g), r2d(tf1_ln2_b),
        tf1_W1, r2d(tf1_b1), tf1_W2, r2d(tf1_b2),
    ]

    out = pl.pallas_call(
        _body,
        out_shape=jax.ShapeDtypeStruct((B * 26, C), jnp.float32),
    )(t0_2, actsf, ss, ss4, *weights)
    return out.reshape(B, 26, C)


# speedup vs baseline: 1.5134x; 1.5134x over previous
import jax
import jax.numpy as jnp
from jax.experimental import pallas as pl

B, T, S, C = 8, 8, 8, 64
S3 = S ** 3
NP = B // 2
NR = NP * S3
C2 = 2 * C
L = T - 1
BL = B * L
NH, DH = 4, 16
N_TF = 2


def _layernorm(x, g, b, eps=1e-5):
    J = jnp.full((C, C), 1.0 / C, jnp.float32)
    mu = jnp.dot(x, J, preferred_element_type=jnp.float32)
    s2 = jnp.dot(x * x, J, preferred_element_type=jnp.float32)
    var = s2 - mu * mu
    return (x - mu) * jax.lax.rsqrt(var + eps) * g + b


def _layernorm2(x, g2, b2, eps=1e-5):
    l1 = jax.lax.broadcasted_iota(jnp.int32, (C2, C2), 0)
    l2 = jax.lax.broadcasted_iota(jnp.int32, (C2, C2), 1)
    J = (l1 // C == l2 // C).astype(jnp.float32) * (1.0 / C)
    mu = jnp.dot(x, J, preferred_element_type=jnp.float32)
    s2 = jnp.dot(x * x, J, preferred_element_type=jnp.float32)
    var = s2 - mu * mu
    return (x - mu) * jax.lax.rsqrt(var + eps) * g2 + b2


def _bd(W):
    Z = jnp.zeros((C, C), jnp.float32)
    return jnp.concatenate(
        [jnp.concatenate([W, Z], axis=1), jnp.concatenate([Z, W], axis=1)],
        axis=0)


def _t2(v):
    return jnp.concatenate([v, v], axis=1)


def _body(t0_ref, acts_ref, ss_ref, *refs):
    out_ref = refs[-1]
    vals = [r[...] for r in refs[:-1]]
    (lin_in_W, lin_in_b,
     conv0_Wl_r, conv0_bl_r, conv0_Wr_r,
     conv1_Wl_r, conv1_bl_r, conv1_Wr_r,
     norm_g_r, norm_b_r,
     act_lin_W, act_lin_b,
     scalar_W, scalar_b) = vals[:14]
    tf = vals[14:]

    lin2_W = jnp.concatenate([lin_in_W, lin_in_W], axis=1)
    lin2_b = _t2(lin_in_b)
    conv0_Wl, conv0_bl, conv0_Wr = _bd(conv0_Wl_r), _t2(conv0_bl_r), _bd(conv0_Wr_r)
    conv1_Wl, conv1_bl, conv1_Wr = _bd(conv1_Wl_r), _t2(conv1_bl_r), _bd(conv1_Wr_r)
    norm_g, norm_b = _t2(norm_g_r), _t2(norm_b_r)

    t0_2 = t0_ref[...]
    acts = acts_ref[...]
    ss = ss_ref[...]
    ss2 = ss.reshape(NP, 2)
    ss4 = jnp.concatenate(
        [jnp.broadcast_to(ss2[:, 0:1], (NP, C)),
         jnp.broadcast_to(ss2[:, 1:2], (NP, C))], axis=1)

    lane = jax.lax.broadcasted_iota(jnp.int32, (NR, C2), 1)
    t0a = jnp.broadcast_to(t0_2[:, 0:1], (NR, C2))
    t0b = jnp.broadcast_to(t0_2[:, 1:2], (NR, C2))
    t0full = jnp.where(lane < C, t0a, t0b)

    rowidx = jax.lax.broadcasted_iota(jnp.int32, (NR, 1), 0)
    sid = rowidx % S3

    pr = jax.lax.broadcasted_iota(jnp.int32, (NP, NR), 0)
    pc = jax.lax.broadcasted_iota(jnp.int32, (NP, NR), 1)
    P = (pr == pc // S3).astype(jnp.float32)

    def bcast_rows(y):
        k = y.shape[1]
        return jnp.broadcast_to(y[:, None, :], (NP, S3, k)).reshape(NR, k)

    maskf_raw = (t0full != 0.0).astype(jnp.float32)
    cnt4 = jnp.dot(P, maskf_raw, preferred_element_type=jnp.float32)
    cnt_rows = bcast_rows(cnt4)
    maskf = jnp.where(jnp.logical_and(cnt_rows == 0.0, sid == 0),
                      1.0, maskf_raw)
    m_rows = jnp.maximum(cnt_rows, 1.0)
    deg = jnp.maximum(maskf * (m_rows - maskf), 1.0)
    md = maskf / deg

    inv_s1 = 1.0 / (S - 1)
    sid512 = jax.lax.broadcasted_iota(jnp.int32, (S3, 1), 0)
    ii = (sid512 // (S * S)).astype(jnp.float32) * inv_s1
    jj = ((sid512 // S) % S).astype(jnp.float32) * inv_s1
    kk = (sid512 % S).astype(jnp.float32) * inv_s1
    base = (ii * lin2_W[0:1, :] + jj * lin2_W[1:2, :] + kk * lin2_W[2:3, :]
            + lin2_b)
    baseN = jnp.broadcast_to(base[None], (NP, S3, C2)).reshape(NR, C2)
    ss_rows = bcast_rows(ss4)
    x = (baseN + (t0full * 0.5) * lin2_W[3:4, :]
         + (ss_rows * (1.0 / S)) * lin2_W[4:5, :])

    for Wl, bl, Wr in ((conv0_Wl, conv0_bl, conv0_Wr),
                       (conv1_Wl, conv1_bl, conv1_Wr)):
        Sx4 = jnp.dot(P, maskf * x, preferred_element_type=jnp.float32)
        SxRows = bcast_rows(Sx4)
        agg = md * (SxRows - x)
        h = (jnp.dot(agg, Wl, preferred_element_type=jnp.float32) + bl
             + jnp.dot(x, Wr, preferred_element_type=jnp.float32))
        x = _layernorm2(jnp.maximum(h, 0.0), norm_g, norm_b)

    full = x * maskf

    r2 = jax.lax.broadcasted_iota(jnp.int32, (24, S3), 0)
    c2 = jax.lax.broadcasted_iota(jnp.int32, (24, S3), 1)
    sel = jnp.where(r2 < 8, c2 // (S * S), jnp.where(r2 < 16, (c2 // S) % S, c2 % S))
    poolW = (sel == (r2 % 8)).astype(jnp.float32)

    a = jnp.dot(acts, act_lin_W, preferred_element_type=jnp.float32) + act_lin_b
    br = jax.lax.broadcasted_iota(jnp.int32, (BL, BL), 0)
    bc = jax.lax.broadcasted_iota(jnp.int32, (BL, BL), 1)
    blockmask = (br // L == bc // L).astype(jnp.float32)
    inv_sqrt_dh = 1.0 / 4.0
    for l in range(N_TF):
        (ln1_g, ln1_b, Wq, bq, Wk, bk, Wv, bv, Wo, bo,
         ln2_g, ln2_b, W1, b1, W2, b2) = tf[l * 16:(l + 1) * 16]
        xa = _layernorm(a, ln1_g, ln1_b)
        q = jnp.dot(xa, Wq, preferred_element_type=jnp.float32) + bq
        k = jnp.dot(xa, Wk, preferred_element_type=jnp.float32) + bk
        v = jnp.dot(xa, Wv, preferred_element_type=jnp.float32) + bv
        outs = []
        for hh in range(NH):
            qh = q[:, hh * DH:(hh + 1) * DH]
            kh = k[:, hh * DH:(hh + 1) * DH]
            vh = v[:, hh * DH:(hh + 1) * DH]
            sc = jax.lax.dot_general(
                qh, kh, (((1,), (1,)), ((), ())),
                preferred_element_type=jnp.float32) * inv_sqrt_dh
            sc = sc - jnp.max(sc, axis=-1, keepdims=True)
            e = jnp.exp(sc) * blockmask
            att = e / jnp.sum(e, axis=-1, keepdims=True)
            outs.append(jnp.dot(att, vh, preferred_element_type=jnp.float32))
        o = jnp.concatenate(outs, axis=1)
        a = a + jnp.dot(o, Wo, preferred_element_type=jnp.float32) + bo
        h2 = _layernorm(a, ln2_g, ln2_b)
        ff = jnp.maximum(
            jnp.dot(h2, W1, preferred_element_type=jnp.float32) + b1, 0.0)
        a = a + jnp.dot(ff, W2, preferred_element_type=jnp.float32) + b2

    ar = jax.lax.broadcasted_iota(jnp.int32, (B, BL), 0)
    ac = jax.lax.broadcasted_iota(jnp.int32, (B, BL), 1)
    Pact = (ar == ac // L).astype(jnp.float32) * (1.0 / L)
    act_emb = jnp.dot(Pact, a, preferred_element_type=jnp.float32)

    mv_emb = jnp.maximum(ss * scalar_W + scalar_b, 0.0)

    for k in range(NP):
        pooled = jnp.dot(poolW, full[k * S3:(k + 1) * S3, :],
                         preferred_element_type=jnp.float32) * (1.0 / 64.0)
        b0, b1 = 2 * k, 2 * k + 1
        out_ref[b0 * 26:b0 * 26 + 24, :] = pooled[:, :C]
        out_ref[b1 * 26:b1 * 26 + 24, :] = pooled[:, C:]
    for b in range(B):
        out_ref[b * 26 + 24:b * 26 + 25, :] = act_emb[b:b + 1, :]
        out_ref[b * 26 + 25:b * 26 + 26, :] = mv_emb[b:b + 1, :]


def kernel(xx, ss, lin_in_W, lin_in_b, conv0_Wl, conv0_bl, conv0_Wr,
           conv1_Wl, conv1_bl, conv1_Wr, norm_g, norm_b,
           act_lin_W, act_lin_b, scalar_W, scalar_b,
           tf0_ln1_g, tf0_ln1_b, tf0_Wq, tf0_bq, tf0_Wk, tf0_bk,
           tf0_Wv, tf0_bv, tf0_Wo, tf0_bo, tf0_ln2_g, tf0_ln2_b,
           tf0_W1, tf0_b1, tf0_W2, tf0_b2,
           tf1_ln1_g, tf1_ln1_b, tf1_Wq, tf1_bq, tf1_Wk, tf1_bk,
           tf1_Wv, tf1_bv, tf1_Wo, tf1_bo, tf1_ln2_g, tf1_ln2_b,
           tf1_W1, tf1_b1, tf1_W2, tf1_b2):
    t0 = xx[:, 0].reshape(B, S3).astype(jnp.float32)
    t0_2 = t0.reshape(NP, 2, S3).transpose(0, 2, 1).reshape(NR, 2)
    actsf = xx[:, 1:].reshape(BL, S3).astype(jnp.float32)

    def r2d(v):
        return v.reshape(1, -1)

    weights = [
        lin_in_W, r2d(lin_in_b),
        conv0_Wl, r2d(conv0_bl), conv0_Wr,
        conv1_Wl, r2d(conv1_bl), conv1_Wr,
        r2d(norm_g), r2d(norm_b),
        act_lin_W, r2d(act_lin_b),
        scalar_W, r2d(scalar_b),
        r2d(tf0_ln1_g), r2d(tf0_ln1_b), tf0_Wq, r2d(tf0_bq), tf0_Wk, r2d(tf0_bk),
        tf0_Wv, r2d(tf0_bv), tf0_Wo, r2d(tf0_bo), r2d(tf0_ln2_g), r2d(tf0_ln2_b),
        tf0_W1, r2d(tf0_b1), tf0_W2, r2d(tf0_b2),
        r2d(tf1_ln1_g), r2d(tf1_ln1_b), tf1_Wq, r2d(tf1_bq), tf1_Wk, r2d(tf1_bk),
        tf1_Wv, r2d(tf1_bv), tf1_Wo, r2d(tf1_bo), r2d(tf1_ln2_g), r2d(tf1_ln2_b),
        tf1_W1, r2d(tf1_b1), tf1_W2, r2d(tf1_b2),
    ]

    out = pl.pallas_call(
        _body,
        out_shape=jax.ShapeDtypeStruct((B * 26, C), jnp.float32),
    )(t0_2, actsf, ss, *weights)
    return out.reshape(B, 26, C)
